# baseline (speedup 1.0000x reference)
import dataclasses

import jax
import jax.numpy as jnp
from jax.experimental import pallas as pl
from jax.experimental.pallas import tpu as pltpu
from jax.experimental.pallas import tpu_sc as plsc

DEPTH = 1000
BATCH = 1024
FEATS = 26
CHUNK_K = 40
CHUNKS_PER_F = DEPTH // CHUNK_K
TOT_CHUNKS = FEATS * CHUNKS_PER_F
NSC = 32
LANES = 16

_MESH = plsc.VectorSubcoreMesh(core_axis_name="c", subcore_axis_name="s")


def _sc_body(idx_hbm, out_hbm, buf, idxv, sem):
    c = jax.lax.axis_index("c")
    s = jax.lax.axis_index("s")
    t = c * 16 + s

    @pl.loop(0, CHUNK_K)
    def _rows(r):
        @pl.loop(0, BATCH // LANES)
        def _groups(g):
            buf.at[r, pl.ds(g * LANES, LANES)][...] = jnp.zeros(
                (LANES,), jnp.float32)

    ones_v = jnp.ones((LANES,), jnp.float32)
    zeros_v = jnp.zeros((LANES,), jnp.float32)
    lane_iota = jax.lax.iota(jnp.int32, LANES)

    @pl.loop(0, (TOT_CHUNKS + NSC - 1) // NSC)
    def _chunks(i):
        m = t + i * NSC

        @pl.when(m < TOT_CHUNKS)
        def _do():
            f = m // CHUNKS_PER_F
            k0 = (m % CHUNKS_PER_F) * CHUNK_K

            pltpu.async_copy(idx_hbm.at[f], idxv, sem).wait()

            def _scatter(vals):
                @pl.loop(0, BATCH // LANES)
                def _g(g):
                    v = idxv.at[pl.ds(g * LANES, LANES)][...]
                    rows = v - k0
                    cols = g * LANES + lane_iota
                    mask = (v >= k0) & (v < k0 + CHUNK_K)
                    plsc.store_scatter(buf, [rows, cols], vals, mask=mask)

            _scatter(ones_v)
            pltpu.async_copy(
                buf, out_hbm.at[f, pl.ds(k0, CHUNK_K), :], sem).wait()
            _scatter(zeros_v)


def kernel(indices):
    idx_t = indices.T

    @pl.kernel(
        out_type=jax.ShapeDtypeStruct((FEATS, DEPTH, BATCH), jnp.float32),
        mesh=_MESH,
        compiler_params=dataclasses.replace(
            pltpu.CompilerParams(), needs_layout_passes=False),
        scratch_types=[
            pltpu.VMEM((CHUNK_K, BATCH), jnp.float32),
            pltpu.VMEM((BATCH,), jnp.int32),
            pltpu.SemaphoreType.DMA,
        ],
    )
    def _run(idx_hbm, out_hbm, buf, idxv, sem):
        _sc_body(idx_hbm, out_hbm, buf, idxv, sem)

    out_t = _run(idx_t)
    return jnp.transpose(out_t, (2, 0, 1))


# speedup vs baseline: 3.1087x; 3.1087x over previous
import jax
import jax.numpy as jnp
from jax.experimental import pallas as pl

DEPTH = 1000
BATCH = 1024
FEATS = 26
def _onehot_t_block(idx_ref, out_ref):
    idx = idx_ref[...]
    k = jax.lax.broadcasted_iota(jnp.int32, (1, DEPTH, BATCH), 1)
    out_ref[...] = (k == idx).astype(jnp.float32)


def kernel(indices):
    idx_t = indices.T.reshape(FEATS, 1, BATCH)
    out_t = pl.pallas_call(
        _onehot_t_block,
        grid=(FEATS,),
        in_specs=[pl.BlockSpec((1, 1, BATCH), lambda i: (i, 0, 0))],
        out_specs=pl.BlockSpec((1, DEPTH, BATCH), lambda i: (i, 0, 0)),
        out_shape=jax.ShapeDtypeStruct((FEATS, DEPTH, BATCH), jnp.float32),
    )(idx_t)
    return jnp.transpose(out_t, (2, 0, 1))
